# baseline (speedup 1.0000x reference)
import functools

import jax
import jax.numpy as jnp
from jax import lax
from jax.experimental import pallas as pl
from jax.experimental.pallas import tpu as pltpu
from jax.experimental.pallas import tpu_sc as plsc

D = 128
KP = 56
NW = 32
NBUF = 8
K_AHEAD = 6
RB = 32


def _gather_kernel(table_hbm, idx_hbm, out_hbm, idx_v, rows_v, gsem, wsem, *,
                   b_per_w, n_chunks, slabs_per_w):
    wid = lax.axis_index("s") * 2 + lax.axis_index("c")
    base = wid * b_per_w
    slab0 = wid * slabs_per_w
    pltpu.sync_copy(idx_hbm.at[pl.ds(base, b_per_w)], idx_v)

    def gather_copy(g, b):
        return pltpu.make_async_copy(
            table_hbm.at[idx_v.at[pl.ds(g * KP, KP)]],
            rows_v.at[b], gsem.at[b])

    def write_copy(g, b):
        return pltpu.make_async_copy(
            rows_v.at[b], out_hbm.at[slab0 + g], wsem.at[b])

    n_outer = n_chunks // NBUF

    for g in range(K_AHEAD):
        gather_copy(g, g % NBUF).start()

    def step(go, bi, issue_gather, wait_write):
        g = go * NBUF + bi
        j = g + K_AHEAD
        bj = (bi + K_AHEAD) % NBUF
        if issue_gather:
            if wait_write:
                write_copy(j - NBUF, bj).wait()
            gather_copy(j, bj).start()
        gather_copy(g, bi).wait()
        write_copy(g, bi).start()

    for bi in range(NBUF):
        step(0, bi, True, bi >= NBUF - K_AHEAD)

    def body(go, carry):
        for bi in range(NBUF):
            step(go, bi, True, True)
        return carry

    lax.fori_loop(1, n_outer - 1, body, 0)

    for bi in range(NBUF):
        g = (n_outer - 1) * NBUF + bi
        if g + K_AHEAD < n_chunks:
            write_copy(g + K_AHEAD - NBUF, (bi + K_AHEAD) % NBUF).wait()
            gather_copy(g + K_AHEAD, (bi + K_AHEAD) % NBUF).start()
        gather_copy(g, bi).wait()
        write_copy(g, bi).start()

    for bi in range(NBUF):
        write_copy((n_outer - 1) * NBUF + bi, bi).wait()


def _relayout_body(in_ref, out_ref):
    out_ref[...] = in_ref[:, :50, :]


def kernel(params, indices):
    nb, k = indices.shape
    idx = jnp.pad(indices.astype(jnp.int32), ((0, 0), (0, KP - k)))
    b = nb * KP
    idx = idx.reshape(b)
    b_per_w = b // NW
    slabs_per_w = nb // NW
    n_chunks = slabs_per_w

    mesh = plsc.VectorSubcoreMesh(core_axis_name="c", subcore_axis_name="s")
    gather = functools.partial(
        pl.kernel,
        mesh=mesh,
        out_type=jax.ShapeDtypeStruct((nb, KP, D), jnp.float32),
        scratch_types=[
            pltpu.VMEM((b_per_w,), jnp.int32),
            pltpu.VMEM((NBUF, KP, D), jnp.float32),
            pltpu.SemaphoreType.DMA((NBUF,)),
            pltpu.SemaphoreType.DMA((NBUF,)),
        ],
    )(functools.partial(_gather_kernel, b_per_w=b_per_w, n_chunks=n_chunks,
                        slabs_per_w=slabs_per_w))

    fp = gather(params, idx)

    relayout = pl.pallas_call(
        _relayout_body,
        grid=(nb // RB,),
        in_specs=[pl.BlockSpec((RB, KP, D), lambda g: (g, 0, 0))],
        out_specs=pl.BlockSpec((RB, k, D), lambda g: (g, 0, 0)),
        out_shape=jax.ShapeDtypeStruct((nb, k, D), jnp.float32),
    )
    return relayout(fp)


# speedup vs baseline: 1.0055x; 1.0055x over previous
import functools

import jax
import jax.numpy as jnp
from jax import lax
from jax.experimental import pallas as pl
from jax.experimental.pallas import tpu as pltpu
from jax.experimental.pallas import tpu_sc as plsc

D = 128
KP = 56
CHUNK = 2 * KP
NW = 32
NBUF = 8
K_AHEAD = 6
RB = 32


def _gather_kernel(table_hbm, idx_hbm, out_hbm, idx_v, rows_v, gsem, wsem, *,
                   b_per_w, n_chunks):
    wid = lax.axis_index("s") * 2 + lax.axis_index("c")
    base = wid * b_per_w
    pltpu.sync_copy(idx_hbm.at[pl.ds(base, b_per_w)], idx_v)

    def gather_copy(g, b):
        return pltpu.make_async_copy(
            table_hbm.at[idx_v.at[pl.ds(g * CHUNK, CHUNK)]],
            rows_v.at[b], gsem.at[b])

    def write_copy(g, b):
        return pltpu.make_async_copy(
            rows_v.at[b], out_hbm.at[pl.ds(base + g * CHUNK, CHUNK)],
            wsem.at[b])

    n_outer = n_chunks // NBUF

    for g in range(K_AHEAD):
        gather_copy(g, g % NBUF).start()

    def step(go, bi, issue_gather, wait_write):
        g = go * NBUF + bi
        j = g + K_AHEAD
        bj = (bi + K_AHEAD) % NBUF
        if issue_gather:
            if wait_write:
                write_copy(j - NBUF, bj).wait()
            gather_copy(j, bj).start()
        gather_copy(g, bi).wait()
        write_copy(g, bi).start()

    for bi in range(NBUF):
        step(0, bi, True, bi >= NBUF - K_AHEAD)

    def body(go, carry):
        for bi in range(NBUF):
            step(go, bi, True, True)
        return carry

    lax.fori_loop(1, n_outer - 1, body, 0)

    for bi in range(NBUF):
        g = (n_outer - 1) * NBUF + bi
        if g + K_AHEAD < n_chunks:
            write_copy(g + K_AHEAD - NBUF, (bi + K_AHEAD) % NBUF).wait()
            gather_copy(g + K_AHEAD, (bi + K_AHEAD) % NBUF).start()
        gather_copy(g, bi).wait()
        write_copy(g, bi).start()

    for bi in range(NBUF):
        write_copy((n_outer - 1) * NBUF + bi, bi).wait()


def _relayout_body(in_ref, out_ref):
    for s in range(RB):
        out_ref[s] = in_ref[pl.ds(s * KP, 50), :]


def kernel(params, indices):
    nb, k = indices.shape
    idx = jnp.pad(indices.astype(jnp.int32), ((0, 0), (0, KP - k)))
    b = nb * KP
    idx = idx.reshape(b)
    b_per_w = b // NW
    n_chunks = b_per_w // CHUNK

    mesh = plsc.VectorSubcoreMesh(core_axis_name="c", subcore_axis_name="s")
    gather = functools.partial(
        pl.kernel,
        mesh=mesh,
        out_type=jax.ShapeDtypeStruct((b, D), jnp.float32),
        scratch_types=[
            pltpu.VMEM((b_per_w,), jnp.int32),
            pltpu.VMEM((NBUF, CHUNK, D), jnp.float32),
            pltpu.SemaphoreType.DMA((NBUF,)),
            pltpu.SemaphoreType.DMA((NBUF,)),
        ],
    )(functools.partial(_gather_kernel, b_per_w=b_per_w, n_chunks=n_chunks))

    flat = gather(params, idx)

    relayout = pl.pallas_call(
        _relayout_body,
        grid=(nb // RB,),
        in_specs=[pl.BlockSpec((RB * KP, D), lambda g: (g, 0))],
        out_specs=pl.BlockSpec((RB, k, D), lambda g: (g, 0, 0)),
        out_shape=jax.ShapeDtypeStruct((nb, k, D), jnp.float32),
    )
    return relayout(flat)


# speedup vs baseline: 5.6667x; 5.6359x over previous
import functools

import jax
import jax.numpy as jnp
from jax import lax
from jax.experimental import pallas as pl
from jax.experimental.pallas import tpu as pltpu
from jax.experimental.pallas import tpu_sc as plsc

D = 128
CHUNK = 64
NW = 32
NBUF = 10
K_AHEAD = 8
RB = 64


def _gather_kernel(table_hbm, idx_hbm, out_hbm, idx_v, rows_v, gsem, wsem, *,
                   b_per_w, n_chunks):
    wid = lax.axis_index("s") * 2 + lax.axis_index("c")
    base = wid * b_per_w
    pltpu.sync_copy(idx_hbm.at[pl.ds(base, b_per_w)], idx_v)

    def gather_copy(g, b):
        return pltpu.make_async_copy(
            table_hbm.at[idx_v.at[pl.ds(g * CHUNK, CHUNK)]],
            rows_v.at[b], gsem.at[b])

    def write_copy(g, b):
        return pltpu.make_async_copy(
            rows_v.at[b], out_hbm.at[pl.ds(base + g * CHUNK, CHUNK)],
            wsem.at[b])

    n_outer = n_chunks // NBUF

    for g in range(K_AHEAD):
        gather_copy(g, g % NBUF).start()

    def step(go, bi, issue_gather, wait_write):
        g = go * NBUF + bi
        j = g + K_AHEAD
        bj = (bi + K_AHEAD) % NBUF
        if issue_gather:
            if wait_write:
                write_copy(j - NBUF, bj).wait()
            gather_copy(j, bj).start()
        gather_copy(g, bi).wait()
        write_copy(g, bi).start()

    for bi in range(NBUF):
        step(0, bi, True, bi >= NBUF - K_AHEAD)

    def body(go, carry):
        for bi in range(NBUF):
            step(go, bi, True, True)
        return carry

    lax.fori_loop(1, n_outer - 1, body, 0)

    for bi in range(NBUF):
        g = (n_outer - 1) * NBUF + bi
        if g + K_AHEAD < n_chunks:
            write_copy(g + K_AHEAD - NBUF, (bi + K_AHEAD) % NBUF).wait()
            gather_copy(g + K_AHEAD, (bi + K_AHEAD) % NBUF).start()
        gather_copy(g, bi).wait()
        write_copy(g, bi).start()

    for bi in range(NBUF):
        write_copy((n_outer - 1) * NBUF + bi, bi).wait()


def _relayout_body(in_ref, out_ref):
    for s in range(RB):
        out_ref[s] = in_ref[pl.ds(s * 50, 50), :]


def kernel(params, indices):
    nb, k = indices.shape
    b = nb * k
    idx = indices.reshape(b).astype(jnp.int32)
    b_per_w = b // NW
    n_chunks = b_per_w // CHUNK

    mesh = plsc.VectorSubcoreMesh(core_axis_name="c", subcore_axis_name="s")
    gather = functools.partial(
        pl.kernel,
        mesh=mesh,
        out_type=jax.ShapeDtypeStruct((b, D), jnp.float32),
        scratch_types=[
            pltpu.VMEM((b_per_w,), jnp.int32),
            pltpu.VMEM((NBUF, CHUNK, D), jnp.float32),
            pltpu.SemaphoreType.DMA((NBUF,)),
            pltpu.SemaphoreType.DMA((NBUF,)),
        ],
    )(functools.partial(_gather_kernel, b_per_w=b_per_w, n_chunks=n_chunks))

    flat = gather(params, idx)

    relayout = pl.pallas_call(
        _relayout_body,
        grid=(nb // RB,),
        in_specs=[pl.BlockSpec((RB * k, D), lambda g: (g, 0))],
        out_specs=pl.BlockSpec((RB, k, D), lambda g: (g, 0, 0)),
        out_shape=jax.ShapeDtypeStruct((nb, k, D), jnp.float32),
    )
    return relayout(flat)


# speedup vs baseline: 5.9042x; 1.0419x over previous
import functools

import jax
import jax.numpy as jnp
from jax import lax
from jax.experimental import pallas as pl
from jax.experimental.pallas import tpu as pltpu
from jax.experimental.pallas import tpu_sc as plsc

D = 128
CHUNK = 64
NW = 32
NBUF = 10
K_AHEAD = 8
RB = 128


def _gather_kernel(table_hbm, idx_hbm, out_hbm, idx_v, rows_v, gsem, wsem, *,
                   b_per_w, n_chunks):
    wid = lax.axis_index("s") * 2 + lax.axis_index("c")
    base = wid * b_per_w
    pltpu.sync_copy(idx_hbm.at[pl.ds(base, b_per_w)], idx_v)

    def gather_copy(g, b):
        return pltpu.make_async_copy(
            table_hbm.at[idx_v.at[pl.ds(g * CHUNK, CHUNK)]],
            rows_v.at[b], gsem.at[b])

    def write_copy(g, b):
        return pltpu.make_async_copy(
            rows_v.at[b], out_hbm.at[pl.ds(base + g * CHUNK, CHUNK)],
            wsem.at[b])

    n_outer = n_chunks // NBUF

    for g in range(K_AHEAD):
        gather_copy(g, g % NBUF).start()

    def step(go, bi, issue_gather, wait_write):
        g = go * NBUF + bi
        j = g + K_AHEAD
        bj = (bi + K_AHEAD) % NBUF
        if issue_gather:
            if wait_write:
                write_copy(j - NBUF, bj).wait()
            gather_copy(j, bj).start()
        gather_copy(g, bi).wait()
        write_copy(g, bi).start()

    for bi in range(NBUF):
        step(0, bi, True, bi >= NBUF - K_AHEAD)

    def body(go, carry):
        for bi in range(NBUF):
            step(go, bi, True, True)
        return carry

    lax.fori_loop(1, n_outer - 1, body, 0)

    for bi in range(NBUF):
        g = (n_outer - 1) * NBUF + bi
        if g + K_AHEAD < n_chunks:
            write_copy(g + K_AHEAD - NBUF, (bi + K_AHEAD) % NBUF).wait()
            gather_copy(g + K_AHEAD, (bi + K_AHEAD) % NBUF).start()
        gather_copy(g, bi).wait()
        write_copy(g, bi).start()

    for bi in range(NBUF):
        write_copy((n_outer - 1) * NBUF + bi, bi).wait()


def _relayout_body(in_ref, out_ref):
    for s in range(RB):
        out_ref[s] = in_ref[pl.ds(s * 50, 50), :]


def kernel(params, indices):
    nb, k = indices.shape
    b = nb * k
    idx = indices.reshape(b).astype(jnp.int32)
    b_per_w = b // NW
    n_chunks = b_per_w // CHUNK

    mesh = plsc.VectorSubcoreMesh(core_axis_name="c", subcore_axis_name="s")
    gather = functools.partial(
        pl.kernel,
        mesh=mesh,
        out_type=jax.ShapeDtypeStruct((b, D), jnp.float32),
        scratch_types=[
            pltpu.VMEM((b_per_w,), jnp.int32),
            pltpu.VMEM((NBUF, CHUNK, D), jnp.float32),
            pltpu.SemaphoreType.DMA((NBUF,)),
            pltpu.SemaphoreType.DMA((NBUF,)),
        ],
    )(functools.partial(_gather_kernel, b_per_w=b_per_w, n_chunks=n_chunks))

    flat = gather(params, idx)

    relayout = pl.pallas_call(
        _relayout_body,
        grid=(nb // RB,),
        in_specs=[pl.BlockSpec((RB * k, D), lambda g: (g, 0))],
        out_specs=pl.BlockSpec((RB, k, D), lambda g: (g, 0, 0)),
        out_shape=jax.ShapeDtypeStruct((nb, k, D), jnp.float32),
    )
    return relayout(flat)


# speedup vs baseline: 5.9446x; 1.0068x over previous
import functools

import jax
import jax.numpy as jnp
from jax import lax
from jax.experimental import pallas as pl
from jax.experimental.pallas import tpu as pltpu
from jax.experimental.pallas import tpu_sc as plsc

D = 128
CHUNK = 64
NW = 32
NBUF = 10
K_AHEAD = 8
RB = 256


def _gather_kernel(table_hbm, idx_hbm, out_hbm, idx_v, rows_v, gsem, wsem, *,
                   b_per_w, n_chunks):
    wid = lax.axis_index("s") * 2 + lax.axis_index("c")
    base = wid * b_per_w
    pltpu.sync_copy(idx_hbm.at[pl.ds(base, b_per_w)], idx_v)

    def gather_copy(g, b):
        return pltpu.make_async_copy(
            table_hbm.at[idx_v.at[pl.ds(g * CHUNK, CHUNK)]],
            rows_v.at[b], gsem.at[b])

    def write_copy(g, b):
        return pltpu.make_async_copy(
            rows_v.at[b], out_hbm.at[pl.ds(base + g * CHUNK, CHUNK)],
            wsem.at[b])

    n_outer = n_chunks // NBUF

    for g in range(K_AHEAD):
        gather_copy(g, g % NBUF).start()

    def step(go, bi, issue_gather, wait_write):
        g = go * NBUF + bi
        j = g + K_AHEAD
        bj = (bi + K_AHEAD) % NBUF
        if issue_gather:
            if wait_write:
                write_copy(j - NBUF, bj).wait()
            gather_copy(j, bj).start()
        gather_copy(g, bi).wait()
        write_copy(g, bi).start()

    for bi in range(NBUF):
        step(0, bi, True, bi >= NBUF - K_AHEAD)

    def body(go, carry):
        for bi in range(NBUF):
            step(go, bi, True, True)
        return carry

    lax.fori_loop(1, n_outer - 1, body, 0)

    for bi in range(NBUF):
        g = (n_outer - 1) * NBUF + bi
        if g + K_AHEAD < n_chunks:
            write_copy(g + K_AHEAD - NBUF, (bi + K_AHEAD) % NBUF).wait()
            gather_copy(g + K_AHEAD, (bi + K_AHEAD) % NBUF).start()
        gather_copy(g, bi).wait()
        write_copy(g, bi).start()

    for bi in range(NBUF):
        write_copy((n_outer - 1) * NBUF + bi, bi).wait()


def _relayout_body(in_ref, out_ref):
    for s in range(RB):
        out_ref[s] = in_ref[pl.ds(s * 50, 50), :]


def kernel(params, indices):
    nb, k = indices.shape
    b = nb * k
    idx = indices.reshape(b).astype(jnp.int32)
    b_per_w = b // NW
    n_chunks = b_per_w // CHUNK

    mesh = plsc.VectorSubcoreMesh(core_axis_name="c", subcore_axis_name="s")
    gather = functools.partial(
        pl.kernel,
        mesh=mesh,
        out_type=jax.ShapeDtypeStruct((b, D), jnp.float32),
        scratch_types=[
            pltpu.VMEM((b_per_w,), jnp.int32),
            pltpu.VMEM((NBUF, CHUNK, D), jnp.float32),
            pltpu.SemaphoreType.DMA((NBUF,)),
            pltpu.SemaphoreType.DMA((NBUF,)),
        ],
    )(functools.partial(_gather_kernel, b_per_w=b_per_w, n_chunks=n_chunks))

    flat = gather(params, idx)

    relayout = pl.pallas_call(
        _relayout_body,
        grid=(nb // RB,),
        in_specs=[pl.BlockSpec((RB * k, D), lambda g: (g, 0))],
        out_specs=pl.BlockSpec((RB, k, D), lambda g: (g, 0, 0)),
        out_shape=jax.ShapeDtypeStruct((nb, k, D), jnp.float32),
    )
    return relayout(flat)


# speedup vs baseline: 6.0003x; 1.0094x over previous
import functools

import jax
import jax.numpy as jnp
from jax import lax
from jax.experimental import pallas as pl
from jax.experimental.pallas import tpu as pltpu
from jax.experimental.pallas import tpu_sc as plsc

D = 128
CHUNK = 64
NW = 32
NBUF = 10
K_AHEAD = 8
RB = 512


def _gather_kernel(table_hbm, idx_hbm, out_hbm, idx_v, rows_v, gsem, wsem, *,
                   b_per_w, n_chunks):
    wid = lax.axis_index("s") * 2 + lax.axis_index("c")
    base = wid * b_per_w
    pltpu.sync_copy(idx_hbm.at[pl.ds(base, b_per_w)], idx_v)

    def gather_copy(g, b):
        return pltpu.make_async_copy(
            table_hbm.at[idx_v.at[pl.ds(g * CHUNK, CHUNK)]],
            rows_v.at[b], gsem.at[b])

    def write_copy(g, b):
        return pltpu.make_async_copy(
            rows_v.at[b], out_hbm.at[pl.ds(base + g * CHUNK, CHUNK)],
            wsem.at[b])

    n_outer = n_chunks // NBUF

    for g in range(K_AHEAD):
        gather_copy(g, g % NBUF).start()

    def step(go, bi, issue_gather, wait_write):
        g = go * NBUF + bi
        j = g + K_AHEAD
        bj = (bi + K_AHEAD) % NBUF
        if issue_gather:
            if wait_write:
                write_copy(j - NBUF, bj).wait()
            gather_copy(j, bj).start()
        gather_copy(g, bi).wait()
        write_copy(g, bi).start()

    for bi in range(NBUF):
        step(0, bi, True, bi >= NBUF - K_AHEAD)

    def body(go, carry):
        for bi in range(NBUF):
            step(go, bi, True, True)
        return carry

    lax.fori_loop(1, n_outer - 1, body, 0)

    for bi in range(NBUF):
        g = (n_outer - 1) * NBUF + bi
        if g + K_AHEAD < n_chunks:
            write_copy(g + K_AHEAD - NBUF, (bi + K_AHEAD) % NBUF).wait()
            gather_copy(g + K_AHEAD, (bi + K_AHEAD) % NBUF).start()
        gather_copy(g, bi).wait()
        write_copy(g, bi).start()

    for bi in range(NBUF):
        write_copy((n_outer - 1) * NBUF + bi, bi).wait()


def _relayout_body(in_ref, out_ref):
    for s in range(RB):
        out_ref[s] = in_ref[pl.ds(s * 50, 50), :]


def kernel(params, indices):
    nb, k = indices.shape
    b = nb * k
    idx = indices.reshape(b).astype(jnp.int32)
    b_per_w = b // NW
    n_chunks = b_per_w // CHUNK

    mesh = plsc.VectorSubcoreMesh(core_axis_name="c", subcore_axis_name="s")
    gather = functools.partial(
        pl.kernel,
        mesh=mesh,
        out_type=jax.ShapeDtypeStruct((b, D), jnp.float32),
        scratch_types=[
            pltpu.VMEM((b_per_w,), jnp.int32),
            pltpu.VMEM((NBUF, CHUNK, D), jnp.float32),
            pltpu.SemaphoreType.DMA((NBUF,)),
            pltpu.SemaphoreType.DMA((NBUF,)),
        ],
    )(functools.partial(_gather_kernel, b_per_w=b_per_w, n_chunks=n_chunks))

    flat = gather(params, idx)

    relayout = pl.pallas_call(
        _relayout_body,
        grid=(nb // RB,),
        in_specs=[pl.BlockSpec((RB * k, D), lambda g: (g, 0))],
        out_specs=pl.BlockSpec((RB, k, D), lambda g: (g, 0, 0)),
        out_shape=jax.ShapeDtypeStruct((nb, k, D), jnp.float32),
    )
    return relayout(flat)
